# baseline (speedup 1.0000x reference)
import functools

import jax
import jax.numpy as jnp
from jax import lax
from jax.experimental import pallas as pl
from jax.experimental.pallas import tpu as pltpu
from jax.experimental.pallas import tpu_sc as plsc

NA = 10000
NB = 160000
NM = 500
AF = 133
BF = 147
H = 300
HP = 384
HG = 128

NC = 2
NS = 16
CB = 128
NCH = NB // CB
NAP = 10240
NROWS_SUB = NAP // NS

RB = 1280
GRID_B = NB // RB
RA = 1000
GRID_A = NA // RA

NCH1 = 640
NB1 = NCH1 * 128
NB2 = NB - NB1
CPS1 = NCH1 // 16
CPS2_TAIL = (NCH - NCH1) - 15 * CPS1

_f32 = jnp.float32


def _pair_swap(x):
    up = pltpu.roll(x, x.shape[0] - 1, 0)
    dn = pltpu.roll(x, 1, 0)
    row = lax.broadcasted_iota(jnp.int32, x.shape, 0)
    return jnp.where(row % 2 == 0, up, dn)


def _mm(a, b):
    return jnp.dot(a, b, preferred_element_type=_f32,
                   precision=lax.Precision.HIGHEST)


def _tc_init_body(fb_ref, wi_ref, wh_ref, g_ref):
    g_ref[...] = _mm(jax.nn.relu(_mm(fb_ref[...], wi_ref[...])), wh_ref[...])


def _tc_hop_body(fb_ref, d_ref, g_ref, wi_ref, wh_ref, gn_ref):
    inp = _mm(fb_ref[...], wi_ref[...])
    m = jax.nn.relu(inp + d_ref[...] - _pair_swap(g_ref[...]))
    gn_ref[...] = _mm(m, wh_ref[...])


def _tc_last_body(fb_ref, d_ref, g_ref, wi_ref, m_ref):
    inp = _mm(fb_ref[...], wi_ref[...])
    m_ref[...] = jax.nn.relu(inp + d_ref[...] - _pair_swap(g_ref[...]))


def _tc_head_body(fa_ref, am_ref, mid_ref, woa_ref, wom_ref, bo_ref,
                  wout_ref, bout_ref, out_ref, acc_s, acc_c):
    i = pl.program_id(0)
    h = jax.nn.relu(_mm(fa_ref[...], woa_ref[...])
                    + _mm(am_ref[...], wom_ref[...])
                    + bo_ref[...])
    s = _mm(h, wout_ref[...])
    ids = mid_ref[0, 0, :]
    onehot = (ids[:, None]
              == lax.broadcasted_iota(jnp.int32, (RA, NM), 1)).astype(_f32)
    dn = (((0,), (0,)), ((), ()))
    sc = lax.dot_general(onehot, s, dn, preferred_element_type=_f32)
    cc = lax.dot_general(onehot, jnp.ones((RA, 1), _f32), dn,
                         preferred_element_type=_f32)

    @pl.when(i == 0)
    def _():
        acc_s[...] = sc
        acc_c[...] = cc

    @pl.when(i > 0)
    def _():
        acc_s[...] += sc
        acc_c[...] += cc

    @pl.when(i == GRID_A - 1)
    def _():
        out_ref[...] = acc_s[...] / jnp.maximum(acc_c[...], 1.0) + bout_ref[...]


def _bspec(bs, w):
    return pl.BlockSpec((bs, w), lambda i: (i, 0))


def _wspec(r, c):
    return pl.BlockSpec((r, c), lambda i: (0, 0))


def _fbspec(blk0):
    return pl.BlockSpec((RB, BF), lambda i, b=blk0: (i + b, 0))


def _mk_init(n, blk0):
    return pl.pallas_call(
        _tc_init_body,
        grid=(n // RB,),
        in_specs=[_fbspec(blk0), _wspec(BF, HP), _wspec(HP, HP)],
        out_specs=_bspec(RB, HP),
        out_shape=jax.ShapeDtypeStruct((n, HP), _f32),
    )


def _mk_hop(n, blk0):
    return pl.pallas_call(
        _tc_hop_body,
        grid=(n // RB,),
        in_specs=[_fbspec(blk0), _bspec(RB, HP), _bspec(RB, HP),
                  _wspec(BF, HP), _wspec(HP, HP)],
        out_specs=_bspec(RB, HP),
        out_shape=jax.ShapeDtypeStruct((n, HP), _f32),
    )


def _mk_last(n, blk0):
    return pl.pallas_call(
        _tc_last_body,
        grid=(n // RB,),
        in_specs=[_fbspec(blk0), _bspec(RB, HP), _bspec(RB, HP),
                  _wspec(BF, HP)],
        out_specs=_bspec(RB, HP),
        out_shape=jax.ShapeDtypeStruct((n, HP), _f32),
    )


_tc_init1, _tc_init2 = _mk_init(NB1, 0), _mk_init(NB2, NB1 // RB)
_tc_hop1, _tc_hop2 = _mk_hop(NB1, 0), _mk_hop(NB2, NB1 // RB)
_tc_last1, _tc_last2 = _mk_last(NB1, 0), _mk_last(NB2, NB1 // RB)

_tc_head = pl.pallas_call(
    _tc_head_body,
    grid=(GRID_A,),
    in_specs=[
        _bspec(RA, AF),
        _bspec(RA, HP),
        pl.BlockSpec((1, 1, RA), lambda i: (i, 0, 0)),
        _wspec(AF, H),
        _wspec(HP, H),
        _wspec(1, H),
        _wspec(H, 1),
        _wspec(1, 1),
    ],
    out_specs=pl.BlockSpec((NM, 1), lambda i: (0, 0)),
    out_shape=jax.ShapeDtypeStruct((NM, 1), _f32),
    scratch_shapes=[pltpu.VMEM((NM, 1), _f32), pltpu.VMEM((NM, 1), _f32)],
)


CPS = 80
CHUNK_TAIL = NCH - 15 * CPS


def _ld_desc(g, buf, u, cl, start, off, sem):
    row = (start + cl) * CB
    return pltpu.make_async_copy(
        g.at[pl.ds(row, CB), pl.ds(off, HG)], buf.at[u], sem)


def _st_desc(d, buf, u, cl, start, off, sem):
    row = (start + cl) * CB
    return pltpu.make_async_copy(
        buf.at[u], d.at[pl.ds(row, CB), pl.ds(off, HG)], sem)


def _idx_desc(a_sh, buf, idx, u, cl, sem, to_table):
    vm = buf.at[u]
    tb = a_sh.at[idx.at[cl]]
    return (pltpu.make_async_copy(vm, tb, sem) if to_table
            else pltpu.make_async_copy(tb, vm, sem))


def _sc_scatter_phase(g, dst2, a_sh, buf, idx, lsems, ssems,
                      trip, c0, gb, off, cpsmax):
    start = c0 - gb
    pltpu.sync_copy(dst2.at[pl.ds(c0, cpsmax), :],
                    idx.at[pl.ds(0, cpsmax), :])
    _ld_desc(g, buf, 0, 0, start, off, lsems[0]).start()

    def body(jo, _):
        for u in range(2):
            cl = 2 * jo + u
            active = cl < trip

            @pl.when(active)
            def _():
                _ld_desc(g, buf, u, cl, start, off, lsems[u]).wait()
                _idx_desc(a_sh, buf, idx, u, cl, ssems[u], True).start(
                    add=True)

            @pl.when(active & (cl >= 1))
            def _():
                _idx_desc(a_sh, buf, idx, 1 - u, cl - 1, ssems[1 - u],
                          True).wait()

            @pl.when(active & (cl + 1 < trip))
            def _():
                _ld_desc(g, buf, 1 - u, cl + 1, start, off,
                         lsems[1 - u]).start()
        return _

    lax.fori_loop(0, cpsmax // 2, body, None)
    _idx_desc(a_sh, buf, idx, 1, trip - 1, ssems[1], True).wait()


def _sc_gather_phase(b2a2, d, a_sh, buf, idx, lsems, ssems,
                     trip, c0, db, off, cpsmax):
    start = c0 - db
    pltpu.sync_copy(b2a2.at[pl.ds(c0, cpsmax), :],
                    idx.at[pl.ds(0, cpsmax), :])
    _idx_desc(a_sh, buf, idx, 0, 0, lsems[0], False).start()

    def body(jo, _):
        for u in range(2):
            cl = 2 * jo + u
            active = cl < trip

            @pl.when(active)
            def _():
                _idx_desc(a_sh, buf, idx, u, cl, lsems[u], False).wait()
                _st_desc(d, buf, u, cl, start, off, ssems[u]).start()

            @pl.when(active & (cl >= 1))
            def _():
                _st_desc(d, buf, 1 - u, cl - 1, start, off,
                         ssems[1 - u]).wait()

            @pl.when(active & (cl + 1 < trip))
            def _():
                _idx_desc(a_sh, buf, idx, 1 - u, cl + 1, lsems[1 - u],
                          False).start()
        return _

    lax.fori_loop(0, cpsmax // 2, body, None)
    _st_desc(d, buf, 1, trip - 1, start, off, ssems[1]).wait()


def _sc_zero(zrows, a_sh, s):
    pltpu.sync_copy(zrows, a_sh.at[pl.ds(s * NROWS_SUB, NROWS_SUB), :])


def _sc_load_table(t, a_sh, s, off):
    pltpu.sync_copy(t.at[pl.ds(s * NROWS_SUB, NROWS_SUB), pl.ds(off, HG)],
                    a_sh.at[pl.ds(s * NROWS_SUB, NROWS_SUB), :])


def _sc_copy_out(am, a_sh, s, off):
    pltpu.sync_copy(a_sh.at[pl.ds(s * NROWS_SUB, NROWS_SUB), :],
                    am.at[pl.ds(s * NROWS_SUB, NROWS_SUB), pl.ds(off, HG)])


@functools.cache
def _sc_kernels():
    mesh = plsc.VectorSubcoreMesh(core_axis_name="c", subcore_axis_name="s",
                                  num_cores=NC, num_subcores=NS)
    scratch = [
        pltpu.VMEM_SHARED((NAP, HG), _f32),
        pltpu.VMEM((2, CB, HG), _f32),
        pltpu.VMEM((CPS, 128), jnp.int32),
    ] + [pltpu.SemaphoreType.DMA] * 6

    def groups_of(body):
        c = lax.axis_index("c")

        @pl.when(c == 0)
        def _():
            body(0)
            body(2 * HG)

        @pl.when(c == 1)
        def _():
            body(HG)

    @functools.partial(
        pl.kernel,
        out_type=[jax.ShapeDtypeStruct((NB1, HP), _f32),
                  jax.ShapeDtypeStruct((NAP, HP), _f32)],
        mesh=mesh,
        scratch_types=scratch,
    )
    def sc_hop_a2(g2, tp, dst2, b2a2, d1, t_out, a_sh, buf, idx,
                  l0, l1, l2, s0, s1, s2):
        s = lax.axis_index("s")
        lsems, ssems = (l0, l1, l2), (s0, s1, s2)

        def one_group(off):
            _sc_load_table(tp, a_sh, s, off)
            plsc.subcore_barrier()
            trip = jnp.where(s == NS - 1, CPS2_TAIL, CPS1)
            _sc_scatter_phase(g2, dst2, a_sh, buf, idx, lsems, ssems,
                              trip=trip, c0=NCH1 + CPS1 * s, gb=NCH1,
                              off=off, cpsmax=CPS1)
            plsc.subcore_barrier()
            _sc_copy_out(t_out, a_sh, s, off)
            _sc_gather_phase(b2a2, d1, a_sh, buf, idx, lsems, ssems,
                             trip=CPS1, c0=CPS1 * s, db=0, off=off,
                             cpsmax=CPS1)
            plsc.subcore_barrier()

        groups_of(one_group)

    @functools.partial(
        pl.kernel,
        out_type=jax.ShapeDtypeStruct((NB2, HP), _f32),
        mesh=mesh,
        scratch_types=scratch,
    )
    def sc_hop_b(t_in, b2a2, d2, a_sh, buf, idx, l0, l1, l2, s0, s1, s2):
        s = lax.axis_index("s")
        lsems, ssems = (l0, l1, l2), (s0, s1, s2)

        def one_group(off):
            _sc_load_table(t_in, a_sh, s, off)
            plsc.subcore_barrier()
            trip = jnp.where(s == NS - 1, CPS2_TAIL, CPS1)
            _sc_gather_phase(b2a2, d2, a_sh, buf, idx, lsems, ssems,
                             trip=trip, c0=NCH1 + CPS1 * s, db=NCH1,
                             off=off, cpsmax=CPS1)
            plsc.subcore_barrier()

        groups_of(one_group)

    @functools.partial(
        pl.kernel,
        out_type=jax.ShapeDtypeStruct((NAP, HP), _f32),
        mesh=mesh,
        scratch_types=scratch,
    )
    def sc_seg1(m1, dst2, zrows, t_out, a_sh, buf, idx, l0, l1, l2,
                s0, s1, s2):
        s = lax.axis_index("s")
        lsems, ssems = (l0, l1, l2), (s0, s1, s2)

        def one_group(off):
            _sc_zero(zrows, a_sh, s)
            plsc.subcore_barrier()
            _sc_scatter_phase(m1, dst2, a_sh, buf, idx, lsems, ssems,
                              trip=CPS1, c0=CPS1 * s, gb=0, off=off,
                              cpsmax=CPS1)
            plsc.subcore_barrier()
            _sc_copy_out(t_out, a_sh, s, off)
            plsc.subcore_barrier()

        groups_of(one_group)

    @functools.partial(
        pl.kernel,
        out_type=jax.ShapeDtypeStruct((NAP, HP), _f32),
        mesh=mesh,
        scratch_types=scratch,
    )
    def sc_seg2(m2, t_in, dst2, am, a_sh, buf, idx, l0, l1, l2,
                s0, s1, s2):
        s = lax.axis_index("s")
        lsems, ssems = (l0, l1, l2), (s0, s1, s2)

        def one_group(off):
            _sc_load_table(t_in, a_sh, s, off)
            plsc.subcore_barrier()
            trip = jnp.where(s == NS - 1, CPS2_TAIL, CPS1)
            _sc_scatter_phase(m2, dst2, a_sh, buf, idx, lsems, ssems,
                              trip=trip, c0=NCH1 + CPS1 * s, gb=NCH1,
                              off=off, cpsmax=CPS1)
            plsc.subcore_barrier()
            _sc_copy_out(am, a_sh, s, off)
            plsc.subcore_barrier()

        groups_of(one_group)

    return sc_hop_a2, sc_hop_b, sc_seg1, sc_seg2


def kernel(f_atoms, f_bonds, bond_dst, b2a, mol_ids,
           W_i, W_h, W_o, b_o, W_out, b_out):
    wip = jnp.pad(W_i, ((0, 0), (0, HP - H)))
    whp = jnp.pad(W_h, ((0, HP - H), (0, HP - H)))
    woa = W_o[:AF]
    womp = jnp.pad(W_o[AF:], ((0, HP - H), (0, 0)))
    bo = b_o.reshape(1, H)
    bout = b_out.reshape(1, 1)

    npad = CPS * NS * 128 - NB
    dst2 = jnp.pad(bond_dst, (0, npad)).reshape(CPS * NS, 128)
    b2a2 = jnp.pad(b2a, (0, npad)).reshape(CPS * NS, 128)
    mid3 = mol_ids.reshape(GRID_A, 1, RA)
    zrows = jnp.zeros((NROWS_SUB, HG), _f32)

    sc_hop_a2, sc_hop_b, sc_seg1, sc_seg2 = _sc_kernels()
    g1 = _tc_init1(f_bonds, wip, whp)
    g2 = _tc_init2(f_bonds, wip, whp)
    for t in range(2):
        tp = sc_seg1(g1, dst2, zrows)
        d1, tt = sc_hop_a2(g2, tp, dst2, b2a2)
        d2 = sc_hop_b(tt, b2a2)
        if t == 0:
            g1 = _tc_hop1(f_bonds, d1, g1, wip, whp)
            g2 = _tc_hop2(f_bonds, d2, g2, wip, whp)
        else:
            m1 = _tc_last1(f_bonds, d1, g1, wip)
            m2 = _tc_last2(f_bonds, d2, g2, wip)
    t1 = sc_seg1(m1, dst2, zrows)
    am = sc_seg2(m2, t1, dst2)
    return _tc_head(f_atoms, am, mid3, woa, womp, bo, W_out, bout)


# speedup vs baseline: 1.5171x; 1.5171x over previous
import functools

import jax
import jax.numpy as jnp
from jax import lax
from jax.experimental import pallas as pl
from jax.experimental.pallas import tpu as pltpu
from jax.experimental.pallas import tpu_sc as plsc

NA = 10000
NB = 160000
NM = 500
AF = 133
BF = 147
H = 300
HP = 384
HG = 128

NC = 2
NS = 16
CB = 128
NCH = NB // CB
NAP = 10240
NROWS_SUB = NAP // NS

RB = 1280
GRID_B = NB // RB
RA = 1000
GRID_A = NA // RA

NCH1 = 640
NB1 = NCH1 * 128
NB2 = NB - NB1
CPS1 = NCH1 // 16
CPS2_TAIL = (NCH - NCH1) - 15 * CPS1

_f32 = jnp.float32


def _pair_swap(x):
    up = pltpu.roll(x, x.shape[0] - 1, 0)
    dn = pltpu.roll(x, 1, 0)
    row = lax.broadcasted_iota(jnp.int32, x.shape, 0)
    return jnp.where(row % 2 == 0, up, dn)


def _mm(a, b):
    return jnp.dot(a, b, preferred_element_type=_f32)


def _tc_init_body(fb_ref, wi_ref, wh_ref, g_ref):
    g_ref[...] = _mm(jax.nn.relu(_mm(fb_ref[...], wi_ref[...])), wh_ref[...])


def _tc_hop_body(fb_ref, d_ref, g_ref, wi_ref, wh_ref, gn_ref):
    inp = _mm(fb_ref[...], wi_ref[...])
    m = jax.nn.relu(inp + d_ref[...] - _pair_swap(g_ref[...]))
    gn_ref[...] = _mm(m, wh_ref[...])


def _tc_last_body(fb_ref, d_ref, g_ref, wi_ref, m_ref):
    inp = _mm(fb_ref[...], wi_ref[...])
    m_ref[...] = jax.nn.relu(inp + d_ref[...] - _pair_swap(g_ref[...]))


def _tc_head_body(fa_ref, am_ref, mid_ref, woa_ref, wom_ref, bo_ref,
                  wout_ref, bout_ref, out_ref, acc_s, acc_c):
    i = pl.program_id(0)
    h = jax.nn.relu(_mm(fa_ref[...], woa_ref[...])
                    + _mm(am_ref[...], wom_ref[...])
                    + bo_ref[...])
    s = _mm(h, wout_ref[...])
    ids = mid_ref[0, 0, :]
    onehot = (ids[:, None]
              == lax.broadcasted_iota(jnp.int32, (RA, NM), 1)).astype(_f32)
    dn = (((0,), (0,)), ((), ()))
    sc = lax.dot_general(onehot, s, dn, preferred_element_type=_f32)
    cc = lax.dot_general(onehot, jnp.ones((RA, 1), _f32), dn,
                         preferred_element_type=_f32)

    @pl.when(i == 0)
    def _():
        acc_s[...] = sc
        acc_c[...] = cc

    @pl.when(i > 0)
    def _():
        acc_s[...] += sc
        acc_c[...] += cc

    @pl.when(i == GRID_A - 1)
    def _():
        out_ref[...] = acc_s[...] / jnp.maximum(acc_c[...], 1.0) + bout_ref[...]


def _bspec(bs, w):
    return pl.BlockSpec((bs, w), lambda i: (i, 0))


def _wspec(r, c):
    return pl.BlockSpec((r, c), lambda i: (0, 0))


def _fbspec(blk0):
    return pl.BlockSpec((RB, BF), lambda i, b=blk0: (i + b, 0))


def _mk_init(n, blk0):
    return pl.pallas_call(
        _tc_init_body,
        grid=(n // RB,),
        in_specs=[_fbspec(blk0), _wspec(BF, HP), _wspec(HP, HP)],
        out_specs=_bspec(RB, HP),
        out_shape=jax.ShapeDtypeStruct((n, HP), _f32),
    )


def _mk_hop(n, blk0):
    return pl.pallas_call(
        _tc_hop_body,
        grid=(n // RB,),
        in_specs=[_fbspec(blk0), _bspec(RB, HP), _bspec(RB, HP),
                  _wspec(BF, HP), _wspec(HP, HP)],
        out_specs=_bspec(RB, HP),
        out_shape=jax.ShapeDtypeStruct((n, HP), _f32),
    )


def _mk_last(n, blk0):
    return pl.pallas_call(
        _tc_last_body,
        grid=(n // RB,),
        in_specs=[_fbspec(blk0), _bspec(RB, HP), _bspec(RB, HP),
                  _wspec(BF, HP)],
        out_specs=_bspec(RB, HP),
        out_shape=jax.ShapeDtypeStruct((n, HP), _f32),
    )


_tc_init1, _tc_init2 = _mk_init(NB1, 0), _mk_init(NB2, NB1 // RB)
_tc_hop1, _tc_hop2 = _mk_hop(NB1, 0), _mk_hop(NB2, NB1 // RB)
_tc_last1, _tc_last2 = _mk_last(NB1, 0), _mk_last(NB2, NB1 // RB)

_tc_head = pl.pallas_call(
    _tc_head_body,
    grid=(GRID_A,),
    in_specs=[
        _bspec(RA, AF),
        _bspec(RA, HP),
        pl.BlockSpec((1, 1, RA), lambda i: (i, 0, 0)),
        _wspec(AF, H),
        _wspec(HP, H),
        _wspec(1, H),
        _wspec(H, 1),
        _wspec(1, 1),
    ],
    out_specs=pl.BlockSpec((NM, 1), lambda i: (0, 0)),
    out_shape=jax.ShapeDtypeStruct((NM, 1), _f32),
    scratch_shapes=[pltpu.VMEM((NM, 1), _f32), pltpu.VMEM((NM, 1), _f32)],
)


CPS = 80
CHUNK_TAIL = NCH - 15 * CPS


def _ld_desc(g, buf, u, cl, start, off, sem):
    row = (start + cl) * CB
    return pltpu.make_async_copy(
        g.at[pl.ds(row, CB), pl.ds(off, HG)], buf.at[u], sem)


def _st_desc(d, buf, u, cl, start, off, sem):
    row = (start + cl) * CB
    return pltpu.make_async_copy(
        buf.at[u], d.at[pl.ds(row, CB), pl.ds(off, HG)], sem)


def _idx_desc(a_sh, buf, idx, u, cl, sem, to_table):
    vm = buf.at[u]
    tb = a_sh.at[idx.at[cl]]
    return (pltpu.make_async_copy(vm, tb, sem) if to_table
            else pltpu.make_async_copy(tb, vm, sem))


def _sc_scatter_phase(g, dst2, a_sh, buf, idx, lsems, ssems,
                      trip, c0, gb, off, cpsmax):
    start = c0 - gb
    pltpu.sync_copy(dst2.at[pl.ds(c0, cpsmax), :],
                    idx.at[pl.ds(0, cpsmax), :])
    _ld_desc(g, buf, 0, 0, start, off, lsems[0]).start()

    def body(jo, _):
        for u in range(2):
            cl = 2 * jo + u
            active = cl < trip

            @pl.when(active)
            def _():
                _ld_desc(g, buf, u, cl, start, off, lsems[u]).wait()
                _idx_desc(a_sh, buf, idx, u, cl, ssems[u], True).start(
                    add=True)

            @pl.when(active & (cl >= 1))
            def _():
                _idx_desc(a_sh, buf, idx, 1 - u, cl - 1, ssems[1 - u],
                          True).wait()

            @pl.when(active & (cl + 1 < trip))
            def _():
                _ld_desc(g, buf, 1 - u, cl + 1, start, off,
                         lsems[1 - u]).start()
        return _

    lax.fori_loop(0, cpsmax // 2, body, None)
    _idx_desc(a_sh, buf, idx, 1, trip - 1, ssems[1], True).wait()


def _sc_gather_phase(b2a2, d, a_sh, buf, idx, lsems, ssems,
                     trip, c0, db, off, cpsmax):
    start = c0 - db
    pltpu.sync_copy(b2a2.at[pl.ds(c0, cpsmax), :],
                    idx.at[pl.ds(0, cpsmax), :])
    _idx_desc(a_sh, buf, idx, 0, 0, lsems[0], False).start()

    def body(jo, _):
        for u in range(2):
            cl = 2 * jo + u
            active = cl < trip

            @pl.when(active)
            def _():
                _idx_desc(a_sh, buf, idx, u, cl, lsems[u], False).wait()
                _st_desc(d, buf, u, cl, start, off, ssems[u]).start()

            @pl.when(active & (cl >= 1))
            def _():
                _st_desc(d, buf, 1 - u, cl - 1, start, off,
                         ssems[1 - u]).wait()

            @pl.when(active & (cl + 1 < trip))
            def _():
                _idx_desc(a_sh, buf, idx, 1 - u, cl + 1, lsems[1 - u],
                          False).start()
        return _

    lax.fori_loop(0, cpsmax // 2, body, None)
    _st_desc(d, buf, 1, trip - 1, start, off, ssems[1]).wait()


def _sc_zero(zrows, a_sh, s):
    pltpu.sync_copy(zrows, a_sh.at[pl.ds(s * NROWS_SUB, NROWS_SUB), :])


def _sc_load_table(t, a_sh, s, off):
    pltpu.sync_copy(t.at[pl.ds(s * NROWS_SUB, NROWS_SUB), pl.ds(off, HG)],
                    a_sh.at[pl.ds(s * NROWS_SUB, NROWS_SUB), :])


def _sc_copy_out(am, a_sh, s, off):
    pltpu.sync_copy(a_sh.at[pl.ds(s * NROWS_SUB, NROWS_SUB), :],
                    am.at[pl.ds(s * NROWS_SUB, NROWS_SUB), pl.ds(off, HG)])


@functools.cache
def _sc_kernels():
    mesh = plsc.VectorSubcoreMesh(core_axis_name="c", subcore_axis_name="s",
                                  num_cores=NC, num_subcores=NS)
    scratch = [
        pltpu.VMEM_SHARED((NAP, HG), _f32),
        pltpu.VMEM((2, CB, HG), _f32),
        pltpu.VMEM((CPS, 128), jnp.int32),
    ] + [pltpu.SemaphoreType.DMA] * 6

    def groups_of(body):
        c = lax.axis_index("c")

        @pl.when(c == 0)
        def _():
            body(0)
            body(2 * HG)

        @pl.when(c == 1)
        def _():
            body(HG)

    @functools.partial(
        pl.kernel,
        out_type=[jax.ShapeDtypeStruct((NB1, HP), _f32),
                  jax.ShapeDtypeStruct((NAP, HP), _f32)],
        mesh=mesh,
        scratch_types=scratch,
    )
    def sc_hop_a2(g2, tp, dst2, b2a2, d1, t_out, a_sh, buf, idx,
                  l0, l1, l2, s0, s1, s2):
        s = lax.axis_index("s")
        lsems, ssems = (l0, l1, l2), (s0, s1, s2)

        def one_group(off):
            _sc_load_table(tp, a_sh, s, off)
            plsc.subcore_barrier()
            trip = jnp.where(s == NS - 1, CPS2_TAIL, CPS1)
            _sc_scatter_phase(g2, dst2, a_sh, buf, idx, lsems, ssems,
                              trip=trip, c0=NCH1 + CPS1 * s, gb=NCH1,
                              off=off, cpsmax=CPS1)
            plsc.subcore_barrier()
            _sc_copy_out(t_out, a_sh, s, off)
            _sc_gather_phase(b2a2, d1, a_sh, buf, idx, lsems, ssems,
                             trip=CPS1, c0=CPS1 * s, db=0, off=off,
                             cpsmax=CPS1)
            plsc.subcore_barrier()

        groups_of(one_group)

    @functools.partial(
        pl.kernel,
        out_type=jax.ShapeDtypeStruct((NB2, HP), _f32),
        mesh=mesh,
        scratch_types=scratch,
    )
    def sc_hop_b(t_in, b2a2, d2, a_sh, buf, idx, l0, l1, l2, s0, s1, s2):
        s = lax.axis_index("s")
        lsems, ssems = (l0, l1, l2), (s0, s1, s2)

        def one_group(off):
            _sc_load_table(t_in, a_sh, s, off)
            plsc.subcore_barrier()
            trip = jnp.where(s == NS - 1, CPS2_TAIL, CPS1)
            _sc_gather_phase(b2a2, d2, a_sh, buf, idx, lsems, ssems,
                             trip=trip, c0=NCH1 + CPS1 * s, db=NCH1,
                             off=off, cpsmax=CPS1)
            plsc.subcore_barrier()

        groups_of(one_group)

    @functools.partial(
        pl.kernel,
        out_type=jax.ShapeDtypeStruct((NAP, HP), _f32),
        mesh=mesh,
        scratch_types=scratch,
    )
    def sc_seg1(m1, dst2, zrows, t_out, a_sh, buf, idx, l0, l1, l2,
                s0, s1, s2):
        s = lax.axis_index("s")
        lsems, ssems = (l0, l1, l2), (s0, s1, s2)

        def one_group(off):
            _sc_zero(zrows, a_sh, s)
            plsc.subcore_barrier()
            _sc_scatter_phase(m1, dst2, a_sh, buf, idx, lsems, ssems,
                              trip=CPS1, c0=CPS1 * s, gb=0, off=off,
                              cpsmax=CPS1)
            plsc.subcore_barrier()
            _sc_copy_out(t_out, a_sh, s, off)
            plsc.subcore_barrier()

        groups_of(one_group)

    @functools.partial(
        pl.kernel,
        out_type=jax.ShapeDtypeStruct((NAP, HP), _f32),
        mesh=mesh,
        scratch_types=scratch,
    )
    def sc_seg2(m2, t_in, dst2, am, a_sh, buf, idx, l0, l1, l2,
                s0, s1, s2):
        s = lax.axis_index("s")
        lsems, ssems = (l0, l1, l2), (s0, s1, s2)

        def one_group(off):
            _sc_load_table(t_in, a_sh, s, off)
            plsc.subcore_barrier()
            trip = jnp.where(s == NS - 1, CPS2_TAIL, CPS1)
            _sc_scatter_phase(m2, dst2, a_sh, buf, idx, lsems, ssems,
                              trip=trip, c0=NCH1 + CPS1 * s, gb=NCH1,
                              off=off, cpsmax=CPS1)
            plsc.subcore_barrier()
            _sc_copy_out(am, a_sh, s, off)
            plsc.subcore_barrier()

        groups_of(one_group)

    return sc_hop_a2, sc_hop_b, sc_seg1, sc_seg2


def kernel(f_atoms, f_bonds, bond_dst, b2a, mol_ids,
           W_i, W_h, W_o, b_o, W_out, b_out):
    wip = jnp.pad(W_i, ((0, 0), (0, HP - H)))
    whp = jnp.pad(W_h, ((0, HP - H), (0, HP - H)))
    woa = W_o[:AF]
    womp = jnp.pad(W_o[AF:], ((0, HP - H), (0, 0)))
    bo = b_o.reshape(1, H)
    bout = b_out.reshape(1, 1)

    npad = CPS * NS * 128 - NB
    dst2 = jnp.pad(bond_dst, (0, npad)).reshape(CPS * NS, 128)
    b2a2 = jnp.pad(b2a, (0, npad)).reshape(CPS * NS, 128)
    mid3 = mol_ids.reshape(GRID_A, 1, RA)
    zrows = jnp.zeros((NROWS_SUB, HG), _f32)

    sc_hop_a2, sc_hop_b, sc_seg1, sc_seg2 = _sc_kernels()
    g1 = _tc_init1(f_bonds, wip, whp)
    g2 = _tc_init2(f_bonds, wip, whp)
    for t in range(2):
        tp = sc_seg1(g1, dst2, zrows)
        d1, tt = sc_hop_a2(g2, tp, dst2, b2a2)
        d2 = sc_hop_b(tt, b2a2)
        if t == 0:
            g1 = _tc_hop1(f_bonds, d1, g1, wip, whp)
            g2 = _tc_hop2(f_bonds, d2, g2, wip, whp)
        else:
            m1 = _tc_last1(f_bonds, d1, g1, wip)
            m2 = _tc_last2(f_bonds, d2, g2, wip)
    t1 = sc_seg1(m1, dst2, zrows)
    am = sc_seg2(m2, t1, dst2)
    return _tc_head(f_atoms, am, mid3, woa, womp, bo, W_out, bout)
